# baseline (speedup 1.0000x reference)
import jax, jax.numpy as jnp


def kernel(h, e, u, Ws, bs, bn_gamma, bn_beta, edge_index, graph_ids):
    src = edge_index[0]
    dst = edge_index[1]
    N = h.shape[0]
    G = u.shape[0]
    E = e.shape[0]

    def lin(x, i):
        return x @ Ws[i] + bs[i]

    def _bn(x, g, b):
        m = jnp.mean(x, axis=0)
        v = jnp.var(x, axis=0)
        return (x - m) / jnp.sqrt(v + 1e-5) * g + b

    Ah = lin(h, 0)
    Be = lin(e, 1)
    Cu = lin(u, 2)
    Cu_atom = Cu[graph_ids]
    e_new = Ah[src] + Ah[dst] + Be + Cu_atom[src]
    e_new = _bn(e_new, bn_gamma[1], bn_beta[1])
    e_new = jax.nn.relu(e_new)
    e_new = e + e_new

    Eh = lin(h, 4)
    sig = jax.nn.sigmoid(e_new)
    num = jax.ops.segment_sum(sig * Eh[src], dst, num_segments=N)
    den = jax.ops.segment_sum(sig, dst, num_segments=N)
    h1 = num / (den + 1e-6)

    Fu = lin(u, 5)
    h2 = Fu[graph_ids]

    h_new = lin(h, 3) + h1 + h2
    h_new = _bn(h_new, bn_gamma[0], bn_beta[0])
    h_new = jax.nn.relu(h_new)
    h_new = h + h_new

    Gh = lin(h_new, 6)
    He = lin(e_new, 7)
    He_atom = jax.ops.segment_sum(He, dst, num_segments=N)
    He_glob = jax.ops.segment_sum(He_atom, graph_ids, num_segments=G)
    mean_He = He_glob / float(E)
    counts = jax.ops.segment_sum(jnp.ones((N, 1), jnp.float32), graph_ids, num_segments=G)
    mean_Gh = jax.ops.segment_sum(Gh, graph_ids, num_segments=G) / jnp.maximum(counts, 1.0)

    u_new = mean_Gh + mean_He + lin(u, 8)
    u_new = _bn(u_new, bn_gamma[2], bn_beta[2])
    u_new = jax.nn.relu(u_new)
    u_new = u + u_new
    return (h_new, e_new, u_new)


# speedup vs baseline: 2.7941x; 2.7941x over previous
import functools

import jax
import jax.numpy as jnp
from jax import lax
from jax.experimental import pallas as pl
from jax.experimental.pallas import tpu as pltpu
from jax.experimental.pallas import tpu_sc as plsc

_PREC = lax.Precision.HIGHEST

_D = 128
_G = 64
_EB = 1280
_CB = 128
_NW = 32
_NS = 16


def _node_proj_body(h_ref, u_ref, gid_ref, ws_ref, bs_ref,
                    p_ref, ah_ref, eh_ref, dfh_ref, starts_ref,
                    stacc_ref, *, nblk):
    i = pl.program_id(0)

    @pl.when(i == 0)
    def _():
        stacc_ref[...] = jnp.zeros_like(stacc_ref)

    h = h_ref[...]
    u = u_ref[...]
    ah = jnp.dot(h, ws_ref[0], precision=_PREC) + bs_ref[0]
    cu = jnp.dot(u, ws_ref[2], precision=_PREC) + bs_ref[2]
    dh = jnp.dot(h, ws_ref[3], precision=_PREC) + bs_ref[3]
    eh = jnp.dot(h, ws_ref[4], precision=_PREC) + bs_ref[4]
    fu = jnp.dot(u, ws_ref[5], precision=_PREC) + bs_ref[5]
    gid = gid_ref[...]
    giota = lax.broadcasted_iota(jnp.int32, (1, _G), 1)
    oh = (gid == giota).astype(jnp.float32)
    cu_atom = jnp.dot(oh, cu, precision=_PREC)
    fu_atom = jnp.dot(oh, fu, precision=_PREC)
    p_ref[...] = ah + cu_atom
    ah_ref[...] = ah
    eh_ref[...] = eh
    dfh_ref[...] = dh + fu_atom
    lt = (gid < giota).astype(jnp.int32)
    le = (gid <= giota).astype(jnp.int32)
    stacc_ref[0:1, :] += jnp.sum(lt, axis=0, keepdims=True)
    stacc_ref[1:2, :] += jnp.sum(le, axis=0, keepdims=True)

    @pl.when(i == nblk - 1)
    def _():
        starts_ref[...] = stacc_ref[0:2, :]


def _sc_gather_body(src_hbm, dst_hbm, p_hbm, ah_hbm, eh_hbm,
                    q_hbm, r_hbm,
                    src_v, dst_v, bufp, bufa, bufe, sem):
    E = src_hbm.shape[0]
    nchunks = E // _CB
    c = lax.axis_index("c")
    s = lax.axis_index("s")
    w = s * 2 + c
    trips = (nchunks + _NW - 1) // _NW

    def trip(t, carry):
        ci = t * _NW + w

        @pl.when(ci < nchunks)
        def _():
            base = ci * _CB
            pltpu.sync_copy(src_hbm.at[pl.ds(base, _CB)], src_v)
            pltpu.sync_copy(dst_hbm.at[pl.ds(base, _CB)], dst_v)
            pltpu.async_copy(p_hbm.at[src_v], bufp, sem).wait()
            pltpu.async_copy(ah_hbm.at[dst_v], bufa, sem).wait()
            pltpu.async_copy(eh_hbm.at[src_v], bufe, sem).wait()

            def add_row(r, carry2):
                for cc in range(_D // 16):
                    sl = pl.ds(cc * 16, 16)
                    bufp[r, sl] = bufp[r, sl] + bufa[r, sl]
                return carry2

            lax.fori_loop(0, _CB, add_row, 0)
            pltpu.sync_copy(bufp, q_hbm.at[pl.ds(base, _CB)])
            pltpu.sync_copy(bufe, r_hbm.at[pl.ds(base, _CB)])

        return carry

    lax.fori_loop(0, trips, trip, 0)


def _edge_pass1_body(e_ref, q_ref, ws_ref, bs_ref, g_ref, b_ref,
                     pre_ref, ac_ref, acc_ref, *, nblk, E):
    i = pl.program_id(0)

    @pl.when(i == 0)
    def _():
        acc_ref[...] = jnp.zeros_like(acc_ref)

    pre = jnp.dot(e_ref[...], ws_ref[1], precision=_PREC) + bs_ref[1] + q_ref[...]
    pre_ref[...] = pre
    acc_ref[0:1, :] += jnp.sum(pre, axis=0, keepdims=True)
    acc_ref[1:2, :] += jnp.sum(pre * pre, axis=0, keepdims=True)

    @pl.when(i == nblk - 1)
    def _():
        m = acc_ref[0:1, :] / E
        v = acc_ref[1:2, :] / E - m * m
        a = g_ref[1:2, :] * lax.rsqrt(v + 1e-5)
        ac_ref[0:1, :] = a
        ac_ref[1:2, :] = b_ref[1:2, :] - m * a


def _edge_pass2_body(pre_ref, e_ref, r_ref, dst_ref, ac_ref, st_ref,
                     enew_ref, v_ref, sg_ref, cnte_ref,
                     sgacc_ref, cntacc_ref, *, nblk):
    i = pl.program_id(0)

    @pl.when(i == 0)
    def _():
        sgacc_ref[...] = jnp.zeros_like(sgacc_ref)
        cntacc_ref[...] = jnp.zeros_like(cntacc_ref)

    a = ac_ref[0:1, :]
    c = ac_ref[1:2, :]
    en = e_ref[...] + jnp.maximum(pre_ref[...] * a + c, 0.0)
    enew_ref[...] = en
    sig = 1.0 / (1.0 + jnp.exp(-en))
    v_ref[:, 0:_D] = sig * r_ref[...]
    v_ref[:, _D:2 * _D] = sig
    dstb = dst_ref[...]
    ge0 = (dstb >= st_ref[0:1, :]).astype(jnp.float32)
    ge1 = (dstb >= st_ref[1:2, :]).astype(jnp.float32)
    oh = ge0 - ge1
    dn = (((0,), (0,)), ((), ()))
    sgacc_ref[...] += lax.dot_general(oh, en, dn, precision=_PREC)
    cntacc_ref[...] += lax.dot_general(oh, jnp.ones_like(en), dn, precision=_PREC)

    @pl.when(i == nblk - 1)
    def _():
        sg_ref[...] = sgacc_ref[...]
        cnte_ref[...] = cntacc_ref[...]


def _sc_scatter_body(vv_hbm, dst_hbm, nd_hbm, dst_v, vbuf, zbuf, acc, sem):
    del sem
    E = dst_hbm.shape[0]
    N = nd_hbm.shape[0]
    nchunks = E // _CB
    ntiles = N // 8
    tiles_per_sub = ntiles // _NS
    rem = ntiles - tiles_per_sub * _NS
    zrows = zbuf.shape[0]
    nz = (tiles_per_sub * 8) // zrows
    c = lax.axis_index("c")
    s = lax.axis_index("s")
    t0 = s * tiles_per_sub + jnp.minimum(s, rem)
    extra = s < rem
    r0 = t0 * 8

    def zrow(r, carry):
        for cc in range(_D // 16):
            zbuf[r, pl.ds(cc * 16, 16)] = jnp.zeros((16,), jnp.float32)
        return carry

    lax.fori_loop(0, zrows, zrow, 0)

    def zcp(j, carry):
        pltpu.sync_copy(zbuf, acc.at[pl.ds(r0 + j * zrows, zrows)])
        return carry

    lax.fori_loop(0, nz, zcp, 0)

    @pl.when(extra)
    def _():
        pltpu.sync_copy(zbuf.at[pl.ds(0, 8)], acc.at[pl.ds(r0 + nz * zrows, 8)])

    plsc.subcore_barrier()

    trips = (nchunks + _NS - 1) // _NS

    def trip(t, carry):
        ci = t * _NS + s

        @pl.when(ci < nchunks)
        def _():
            base = ci * _CB
            pltpu.sync_copy(dst_hbm.at[pl.ds(base, _CB)], dst_v)
            pltpu.sync_copy(vv_hbm.at[pl.ds(base, _CB), pl.ds(c * _D, _D)], vbuf)
            pltpu.sync_copy(vbuf, acc.at[dst_v], add=True)

        return carry

    lax.fori_loop(0, trips, trip, 0)
    plsc.subcore_barrier()

    def dump(j, carry):
        rr = r0 + j * zrows
        pltpu.sync_copy(acc.at[pl.ds(rr, zrows)],
                        nd_hbm.at[pl.ds(rr, zrows), pl.ds(c * _D, _D)])
        return carry

    lax.fori_loop(0, nz, dump, 0)

    @pl.when(extra)
    def _():
        rr = r0 + nz * zrows
        pltpu.sync_copy(acc.at[pl.ds(rr, 8)],
                        nd_hbm.at[pl.ds(rr, 8), pl.ds(c * _D, _D)])


def _node_pass1_body(nd_ref, dfh_ref, g_ref, b_ref,
                     hpre_ref, ac_ref, acc_ref, *, nblk, N):
    i = pl.program_id(0)

    @pl.when(i == 0)
    def _():
        acc_ref[...] = jnp.zeros_like(acc_ref)

    num = nd_ref[:, 0:_D]
    den = nd_ref[:, _D:2 * _D]
    hpre = dfh_ref[...] + num / (den + 1e-6)
    hpre_ref[...] = hpre
    acc_ref[0:1, :] += jnp.sum(hpre, axis=0, keepdims=True)
    acc_ref[1:2, :] += jnp.sum(hpre * hpre, axis=0, keepdims=True)

    @pl.when(i == nblk - 1)
    def _():
        m = acc_ref[0:1, :] / N
        v = acc_ref[1:2, :] / N - m * m
        a = g_ref[0:1, :] * lax.rsqrt(v + 1e-5)
        ac_ref[0:1, :] = a
        ac_ref[1:2, :] = b_ref[0:1, :] - m * a


def _node_pass2_body(hpre_ref, h_ref, gid_ref, ac_ref,
                     hn_ref, sh_ref, cntn_ref, shacc_ref, cntacc_ref, *, nblk):
    i = pl.program_id(0)

    @pl.when(i == 0)
    def _():
        shacc_ref[...] = jnp.zeros_like(shacc_ref)
        cntacc_ref[...] = jnp.zeros_like(cntacc_ref)

    a = ac_ref[0:1, :]
    c = ac_ref[1:2, :]
    hn = h_ref[...] + jnp.maximum(hpre_ref[...] * a + c, 0.0)
    hn_ref[...] = hn
    gid = gid_ref[...]
    oh = (gid == lax.broadcasted_iota(jnp.int32, (1, _G), 1)).astype(jnp.float32)
    dn = (((0,), (0,)), ((), ()))
    shacc_ref[...] += lax.dot_general(oh, hn, dn, precision=_PREC)
    cntacc_ref[...] += lax.dot_general(oh, jnp.ones_like(hn), dn, precision=_PREC)

    @pl.when(i == nblk - 1)
    def _():
        sh_ref[...] = shacc_ref[...]
        cntn_ref[...] = cntacc_ref[...]


def _global_body(u_ref, sg_ref, cnte_ref, sh_ref, cntn_ref, ws_ref, bs_ref,
                 g_ref, b_ref, un_ref, *, E, G):
    u = u_ref[...]
    cntn = cntn_ref[...]
    mean_gh = (jnp.dot(sh_ref[...], ws_ref[6], precision=_PREC)
               + cntn * bs_ref[6]) / jnp.maximum(cntn, 1.0)
    mean_he = (jnp.dot(sg_ref[...], ws_ref[7], precision=_PREC)
               + cnte_ref[...] * bs_ref[7]) / E
    upre = (mean_gh + mean_he
            + jnp.dot(u, ws_ref[8], precision=_PREC) + bs_ref[8])
    m = jnp.sum(upre, axis=0, keepdims=True) / G
    v = jnp.sum(upre * upre, axis=0, keepdims=True) / G - m * m
    un_ref[...] = u + jnp.maximum(
        (upre - m) * lax.rsqrt(v + 1e-5) * g_ref[2:3, :] + b_ref[2:3, :], 0.0)


def kernel(h, e, u, Ws, bs, bn_gamma, bn_beta, edge_index, graph_ids):
    N, D = h.shape
    E = e.shape[0]
    G = u.shape[0]
    assert D == _D and G == _G and E % _EB == 0 and E % _CB == 0

    src = edge_index[0]
    dst = edge_index[1]
    gid2 = graph_ids.reshape(N, 1)
    dst2 = dst.reshape(E, 1)
    f32 = jnp.float32

    NB = 2000
    nnblk = N // NB
    p_, ah_, eh_, dfh_, starts_ = pl.pallas_call(
        functools.partial(_node_proj_body, nblk=nnblk),
        grid=(nnblk,),
        in_specs=[
            pl.BlockSpec((NB, D), lambda i: (i, 0)),
            pl.BlockSpec((G, D), lambda i: (0, 0)),
            pl.BlockSpec((NB, 1), lambda i: (i, 0)),
            pl.BlockSpec((9, D, D), lambda i: (0, 0, 0)),
            pl.BlockSpec((9, D), lambda i: (0, 0)),
        ],
        out_specs=(
            pl.BlockSpec((NB, D), lambda i: (i, 0)),
            pl.BlockSpec((NB, D), lambda i: (i, 0)),
            pl.BlockSpec((NB, D), lambda i: (i, 0)),
            pl.BlockSpec((NB, D), lambda i: (i, 0)),
            pl.BlockSpec((2, G), lambda i: (0, 0)),
        ),
        out_shape=(
            jax.ShapeDtypeStruct((N, D), f32),
            jax.ShapeDtypeStruct((N, D), f32),
            jax.ShapeDtypeStruct((N, D), f32),
            jax.ShapeDtypeStruct((N, D), f32),
            jax.ShapeDtypeStruct((2, G), jnp.int32),
        ),
        scratch_shapes=[pltpu.VMEM((8, G), jnp.int32)],
    )(h, u, gid2, Ws, bs)

    mesh = plsc.VectorSubcoreMesh(core_axis_name="c", subcore_axis_name="s")
    q_, r_ = pl.kernel(
        _sc_gather_body,
        out_type=(
            jax.ShapeDtypeStruct((E, D), f32),
            jax.ShapeDtypeStruct((E, D), f32),
        ),
        mesh=mesh,
        scratch_types=[
            pltpu.VMEM((_CB,), jnp.int32),
            pltpu.VMEM((_CB,), jnp.int32),
            pltpu.VMEM((_CB, D), f32),
            pltpu.VMEM((_CB, D), f32),
            pltpu.VMEM((_CB, D), f32),
            pltpu.SemaphoreType.DMA,
        ],
    )(src, dst, p_, ah_, eh_)

    nblk = E // _EB
    pre_, ac_ = pl.pallas_call(
        functools.partial(_edge_pass1_body, nblk=nblk, E=float(E)),
        grid=(nblk,),
        in_specs=[
            pl.BlockSpec((_EB, D), lambda i: (i, 0)),
            pl.BlockSpec((_EB, D), lambda i: (i, 0)),
            pl.BlockSpec((9, D, D), lambda i: (0, 0, 0)),
            pl.BlockSpec((9, D), lambda i: (0, 0)),
            pl.BlockSpec((3, D), lambda i: (0, 0)),
            pl.BlockSpec((3, D), lambda i: (0, 0)),
        ],
        out_specs=(
            pl.BlockSpec((_EB, D), lambda i: (i, 0)),
            pl.BlockSpec((2, D), lambda i: (0, 0)),
        ),
        out_shape=(
            jax.ShapeDtypeStruct((E, D), f32),
            jax.ShapeDtypeStruct((2, D), f32),
        ),
        scratch_shapes=[pltpu.VMEM((8, D), f32)],
    )(e, q_, Ws, bs, bn_gamma, bn_beta)

    stf = starts_
    enew_, v_, sg_, cnte_ = pl.pallas_call(
        functools.partial(_edge_pass2_body, nblk=nblk),
        grid=(nblk,),
        in_specs=[
            pl.BlockSpec((_EB, D), lambda i: (i, 0)),
            pl.BlockSpec((_EB, D), lambda i: (i, 0)),
            pl.BlockSpec((_EB, D), lambda i: (i, 0)),
            pl.BlockSpec((_EB, 1), lambda i: (i, 0)),
            pl.BlockSpec((2, D), lambda i: (0, 0)),
            pl.BlockSpec((2, G), lambda i: (0, 0)),
        ],
        out_specs=(
            pl.BlockSpec((_EB, D), lambda i: (i, 0)),
            pl.BlockSpec((_EB, 2 * D), lambda i: (i, 0)),
            pl.BlockSpec((G, D), lambda i: (0, 0)),
            pl.BlockSpec((G, D), lambda i: (0, 0)),
        ),
        out_shape=(
            jax.ShapeDtypeStruct((E, D), f32),
            jax.ShapeDtypeStruct((E, 2 * D), f32),
            jax.ShapeDtypeStruct((G, D), f32),
            jax.ShapeDtypeStruct((G, D), f32),
        ),
        scratch_shapes=[pltpu.VMEM((G, D), f32), pltpu.VMEM((G, D), f32)],
    )(pre_, e, r_, dst2, ac_, stf)

    nd_ = pl.kernel(
        _sc_scatter_body,
        out_type=jax.ShapeDtypeStruct((N, 2 * D), f32),
        mesh=mesh,
        scratch_types=[
            pltpu.VMEM((_CB,), jnp.int32),
            pltpu.VMEM((_CB, D), f32),
            pltpu.VMEM((104, D), f32),
            pltpu.VMEM_SHARED((N, D), f32),
            pltpu.SemaphoreType.DMA,
        ],
    )(v_, dst)

    hpre_, ac0_ = pl.pallas_call(
        functools.partial(_node_pass1_body, nblk=nnblk, N=float(N)),
        grid=(nnblk,),
        in_specs=[
            pl.BlockSpec((NB, 2 * D), lambda i: (i, 0)),
            pl.BlockSpec((NB, D), lambda i: (i, 0)),
            pl.BlockSpec((3, D), lambda i: (0, 0)),
            pl.BlockSpec((3, D), lambda i: (0, 0)),
        ],
        out_specs=(
            pl.BlockSpec((NB, D), lambda i: (i, 0)),
            pl.BlockSpec((2, D), lambda i: (0, 0)),
        ),
        out_shape=(
            jax.ShapeDtypeStruct((N, D), f32),
            jax.ShapeDtypeStruct((2, D), f32),
        ),
        scratch_shapes=[pltpu.VMEM((8, D), f32)],
    )(nd_, dfh_, bn_gamma, bn_beta)

    hn_, sh_, cntn_ = pl.pallas_call(
        functools.partial(_node_pass2_body, nblk=nnblk),
        grid=(nnblk,),
        in_specs=[
            pl.BlockSpec((NB, D), lambda i: (i, 0)),
            pl.BlockSpec((NB, D), lambda i: (i, 0)),
            pl.BlockSpec((NB, 1), lambda i: (i, 0)),
            pl.BlockSpec((2, D), lambda i: (0, 0)),
        ],
        out_specs=(
            pl.BlockSpec((NB, D), lambda i: (i, 0)),
            pl.BlockSpec((G, D), lambda i: (0, 0)),
            pl.BlockSpec((G, D), lambda i: (0, 0)),
        ),
        out_shape=(
            jax.ShapeDtypeStruct((N, D), f32),
            jax.ShapeDtypeStruct((G, D), f32),
            jax.ShapeDtypeStruct((G, D), f32),
        ),
        scratch_shapes=[pltpu.VMEM((G, D), f32), pltpu.VMEM((G, D), f32)],
    )(hpre_, h, gid2, ac0_)

    un_ = pl.pallas_call(
        functools.partial(_global_body, E=float(E), G=float(G)),
        out_shape=jax.ShapeDtypeStruct((G, D), f32),
    )(u, sg_, cnte_, sh_, cntn_, Ws, bs, bn_gamma, bn_beta)

    return (hn_, enew_, un_)


# speedup vs baseline: 3.0649x; 1.0969x over previous
import functools

import jax
import jax.numpy as jnp
from jax import lax
from jax.experimental import pallas as pl
from jax.experimental.pallas import tpu as pltpu
from jax.experimental.pallas import tpu_sc as plsc

_PREC = lax.Precision.HIGHEST

_D = 128
_G = 64
_EB = 1280
_CB = 128
_NW = 32
_NS = 16


def _node_proj_body(h_ref, u_ref, gid_ref, ws_ref, bs_ref,
                    p_ref, ah_ref, eh_ref, dfh_ref, starts_ref,
                    stacc_ref, *, nblk):
    i = pl.program_id(0)

    @pl.when(i == 0)
    def _():
        stacc_ref[...] = jnp.zeros_like(stacc_ref)

    h = h_ref[...]
    u = u_ref[...]
    ah = jnp.dot(h, ws_ref[0], precision=_PREC) + bs_ref[0]
    cu = jnp.dot(u, ws_ref[2], precision=_PREC) + bs_ref[2]
    dh = jnp.dot(h, ws_ref[3], precision=_PREC) + bs_ref[3]
    eh = jnp.dot(h, ws_ref[4], precision=_PREC) + bs_ref[4]
    fu = jnp.dot(u, ws_ref[5], precision=_PREC) + bs_ref[5]
    gid = gid_ref[...]
    giota = lax.broadcasted_iota(jnp.int32, (1, _G), 1)
    oh = (gid == giota).astype(jnp.float32)
    cu_atom = jnp.dot(oh, cu, precision=_PREC)
    fu_atom = jnp.dot(oh, fu, precision=_PREC)
    p_ref[...] = ah + cu_atom
    ah_ref[...] = ah
    eh_ref[...] = eh
    dfh_ref[...] = dh + fu_atom
    lt = (gid < giota).astype(jnp.int32)
    le = (gid <= giota).astype(jnp.int32)
    stacc_ref[0:1, :] += jnp.sum(lt, axis=0, keepdims=True)
    stacc_ref[1:2, :] += jnp.sum(le, axis=0, keepdims=True)

    @pl.when(i == nblk - 1)
    def _():
        starts_ref[...] = stacc_ref[0:2, :]


def _sc_gather_q_body(src_hbm, dst_hbm, p_hbm, ah_hbm,
                      q_hbm,
                      src_v, dst_v, bufp, bufa, sem):
    E = src_hbm.shape[0]
    nchunks = E // _CB
    c = lax.axis_index("c")
    s = lax.axis_index("s")
    w = s * 2 + c
    trips = (nchunks + _NW - 1) // _NW

    def trip(t, carry):
        ci = t * _NW + w

        @pl.when(ci < nchunks)
        def _():
            base = ci * _CB
            pltpu.sync_copy(src_hbm.at[pl.ds(base, _CB)], src_v)
            pltpu.sync_copy(dst_hbm.at[pl.ds(base, _CB)], dst_v)
            cp_p = pltpu.async_copy(p_hbm.at[src_v], bufp, sem)
            cp_a = pltpu.async_copy(ah_hbm.at[dst_v], bufa, sem)
            cp_p.wait()
            cp_a.wait()

            def add_row(r, carry2):
                for cc in range(_D // 16):
                    sl = pl.ds(cc * 16, 16)
                    bufp[r, sl] = bufp[r, sl] + bufa[r, sl]
                return carry2

            lax.fori_loop(0, _CB, add_row, 0)
            pltpu.sync_copy(bufp, q_hbm.at[pl.ds(base, _CB)])

        return carry

    lax.fori_loop(0, trips, trip, 0)


def _sc_gather_r_body(src_hbm, eh_hbm, r_hbm, src_v, bufe, sem):
    E = src_hbm.shape[0]
    nchunks = E // _CB
    c = lax.axis_index("c")
    s = lax.axis_index("s")
    w = s * 2 + c
    trips = (nchunks + _NW - 1) // _NW

    def trip(t, carry):
        ci = t * _NW + w

        @pl.when(ci < nchunks)
        def _():
            base = ci * _CB
            pltpu.sync_copy(src_hbm.at[pl.ds(base, _CB)], src_v)
            pltpu.async_copy(eh_hbm.at[src_v], bufe, sem).wait()
            pltpu.sync_copy(bufe, r_hbm.at[pl.ds(base, _CB)])

        return carry

    lax.fori_loop(0, trips, trip, 0)


def _edge_pass1_body(e_ref, q_ref, ws_ref, bs_ref, g_ref, b_ref,
                     pre_ref, ac_ref, acc_ref, *, nblk, E):
    i = pl.program_id(0)

    @pl.when(i == 0)
    def _():
        acc_ref[...] = jnp.zeros_like(acc_ref)

    pre = jnp.dot(e_ref[...], ws_ref[1], precision=_PREC) + bs_ref[1] + q_ref[...]
    pre_ref[...] = pre
    acc_ref[0:1, :] += jnp.sum(pre, axis=0, keepdims=True)
    acc_ref[1:2, :] += jnp.sum(pre * pre, axis=0, keepdims=True)

    @pl.when(i == nblk - 1)
    def _():
        m = acc_ref[0:1, :] / E
        v = acc_ref[1:2, :] / E - m * m
        a = g_ref[1:2, :] * lax.rsqrt(v + 1e-5)
        ac_ref[0:1, :] = a
        ac_ref[1:2, :] = b_ref[1:2, :] - m * a


def _edge_pass2_body(pre_ref, e_ref, r_ref, dst_ref, ac_ref, st_ref,
                     enew_ref, v_ref, sg_ref, cnte_ref,
                     sgacc_ref, cntacc_ref, *, nblk):
    i = pl.program_id(0)

    @pl.when(i == 0)
    def _():
        sgacc_ref[...] = jnp.zeros_like(sgacc_ref)
        cntacc_ref[...] = jnp.zeros_like(cntacc_ref)

    a = ac_ref[0:1, :]
    c = ac_ref[1:2, :]
    en = e_ref[...] + jnp.maximum(pre_ref[...] * a + c, 0.0)
    enew_ref[...] = en
    sig = 1.0 / (1.0 + jnp.exp(-en))
    v_ref[:, 0:_D] = sig * r_ref[...]
    v_ref[:, _D:2 * _D] = sig
    dstb = dst_ref[...]
    ge0 = (dstb >= st_ref[0:1, :]).astype(jnp.float32)
    ge1 = (dstb >= st_ref[1:2, :]).astype(jnp.float32)
    oh = ge0 - ge1
    dn = (((0,), (0,)), ((), ()))
    sgacc_ref[...] += lax.dot_general(oh, en, dn, precision=_PREC)
    cntacc_ref[...] += lax.dot_general(oh, jnp.ones_like(en), dn, precision=_PREC)

    @pl.when(i == nblk - 1)
    def _():
        sg_ref[...] = sgacc_ref[...]
        cnte_ref[...] = cntacc_ref[...]


def _sc_scatter_body(vv_hbm, dst_hbm, nd_hbm, dst_v, vbuf, zbuf, acc, sem):
    del sem
    E = dst_hbm.shape[0]
    N = nd_hbm.shape[0]
    nchunks = E // _CB
    ntiles = N // 8
    tiles_per_sub = ntiles // _NS
    rem = ntiles - tiles_per_sub * _NS
    zrows = zbuf.shape[0]
    nz = (tiles_per_sub * 8) // zrows
    c = lax.axis_index("c")
    s = lax.axis_index("s")
    t0 = s * tiles_per_sub + jnp.minimum(s, rem)
    extra = s < rem
    r0 = t0 * 8

    def zrow(r, carry):
        for cc in range(_D // 16):
            zbuf[r, pl.ds(cc * 16, 16)] = jnp.zeros((16,), jnp.float32)
        return carry

    lax.fori_loop(0, zrows, zrow, 0)

    def zcp(j, carry):
        pltpu.sync_copy(zbuf, acc.at[pl.ds(r0 + j * zrows, zrows)])
        return carry

    lax.fori_loop(0, nz, zcp, 0)

    @pl.when(extra)
    def _():
        pltpu.sync_copy(zbuf.at[pl.ds(0, 8)], acc.at[pl.ds(r0 + nz * zrows, 8)])

    plsc.subcore_barrier()

    trips = (nchunks + _NS - 1) // _NS

    def trip(t, carry):
        ci = t * _NS + s

        @pl.when(ci < nchunks)
        def _():
            base = ci * _CB
            pltpu.sync_copy(dst_hbm.at[pl.ds(base, _CB)], dst_v)
            pltpu.sync_copy(vv_hbm.at[pl.ds(base, _CB), pl.ds(c * _D, _D)], vbuf)
            pltpu.sync_copy(vbuf, acc.at[dst_v], add=True)

        return carry

    lax.fori_loop(0, trips, trip, 0)
    plsc.subcore_barrier()

    def dump(j, carry):
        rr = r0 + j * zrows
        pltpu.sync_copy(acc.at[pl.ds(rr, zrows)],
                        nd_hbm.at[pl.ds(rr, zrows), pl.ds(c * _D, _D)])
        return carry

    lax.fori_loop(0, nz, dump, 0)

    @pl.when(extra)
    def _():
        rr = r0 + nz * zrows
        pltpu.sync_copy(acc.at[pl.ds(rr, 8)],
                        nd_hbm.at[pl.ds(rr, 8), pl.ds(c * _D, _D)])


def _node_pass1_body(nd_ref, dfh_ref, g_ref, b_ref,
                     hpre_ref, ac_ref, acc_ref, *, nblk, N):
    i = pl.program_id(0)

    @pl.when(i == 0)
    def _():
        acc_ref[...] = jnp.zeros_like(acc_ref)

    num = nd_ref[:, 0:_D]
    den = nd_ref[:, _D:2 * _D]
    hpre = dfh_ref[...] + num / (den + 1e-6)
    hpre_ref[...] = hpre
    acc_ref[0:1, :] += jnp.sum(hpre, axis=0, keepdims=True)
    acc_ref[1:2, :] += jnp.sum(hpre * hpre, axis=0, keepdims=True)

    @pl.when(i == nblk - 1)
    def _():
        m = acc_ref[0:1, :] / N
        v = acc_ref[1:2, :] / N - m * m
        a = g_ref[0:1, :] * lax.rsqrt(v + 1e-5)
        ac_ref[0:1, :] = a
        ac_ref[1:2, :] = b_ref[0:1, :] - m * a


def _node_pass2_body(hpre_ref, h_ref, gid_ref, ac_ref,
                     hn_ref, sh_ref, cntn_ref, shacc_ref, cntacc_ref, *, nblk):
    i = pl.program_id(0)

    @pl.when(i == 0)
    def _():
        shacc_ref[...] = jnp.zeros_like(shacc_ref)
        cntacc_ref[...] = jnp.zeros_like(cntacc_ref)

    a = ac_ref[0:1, :]
    c = ac_ref[1:2, :]
    hn = h_ref[...] + jnp.maximum(hpre_ref[...] * a + c, 0.0)
    hn_ref[...] = hn
    gid = gid_ref[...]
    oh = (gid == lax.broadcasted_iota(jnp.int32, (1, _G), 1)).astype(jnp.float32)
    dn = (((0,), (0,)), ((), ()))
    shacc_ref[...] += lax.dot_general(oh, hn, dn, precision=_PREC)
    cntacc_ref[...] += lax.dot_general(oh, jnp.ones_like(hn), dn, precision=_PREC)

    @pl.when(i == nblk - 1)
    def _():
        sh_ref[...] = shacc_ref[...]
        cntn_ref[...] = cntacc_ref[...]


def _global_body(u_ref, sg_ref, cnte_ref, sh_ref, cntn_ref, ws_ref, bs_ref,
                 g_ref, b_ref, un_ref, *, E, G):
    u = u_ref[...]
    cntn = cntn_ref[...]
    mean_gh = (jnp.dot(sh_ref[...], ws_ref[6], precision=_PREC)
               + cntn * bs_ref[6]) / jnp.maximum(cntn, 1.0)
    mean_he = (jnp.dot(sg_ref[...], ws_ref[7], precision=_PREC)
               + cnte_ref[...] * bs_ref[7]) / E
    upre = (mean_gh + mean_he
            + jnp.dot(u, ws_ref[8], precision=_PREC) + bs_ref[8])
    m = jnp.sum(upre, axis=0, keepdims=True) / G
    v = jnp.sum(upre * upre, axis=0, keepdims=True) / G - m * m
    un_ref[...] = u + jnp.maximum(
        (upre - m) * lax.rsqrt(v + 1e-5) * g_ref[2:3, :] + b_ref[2:3, :], 0.0)


def kernel(h, e, u, Ws, bs, bn_gamma, bn_beta, edge_index, graph_ids):
    N, D = h.shape
    E = e.shape[0]
    G = u.shape[0]
    assert D == _D and G == _G and E % _EB == 0 and E % _CB == 0

    src = edge_index[0]
    dst = edge_index[1]
    gid2 = graph_ids.reshape(N, 1)
    dst2 = dst.reshape(E, 1)
    f32 = jnp.float32

    NB = 2000
    nnblk = N // NB
    p_, ah_, eh_, dfh_, starts_ = pl.pallas_call(
        functools.partial(_node_proj_body, nblk=nnblk),
        grid=(nnblk,),
        in_specs=[
            pl.BlockSpec((NB, D), lambda i: (i, 0)),
            pl.BlockSpec((G, D), lambda i: (0, 0)),
            pl.BlockSpec((NB, 1), lambda i: (i, 0)),
            pl.BlockSpec((9, D, D), lambda i: (0, 0, 0)),
            pl.BlockSpec((9, D), lambda i: (0, 0)),
        ],
        out_specs=(
            pl.BlockSpec((NB, D), lambda i: (i, 0)),
            pl.BlockSpec((NB, D), lambda i: (i, 0)),
            pl.BlockSpec((NB, D), lambda i: (i, 0)),
            pl.BlockSpec((NB, D), lambda i: (i, 0)),
            pl.BlockSpec((2, G), lambda i: (0, 0)),
        ),
        out_shape=(
            jax.ShapeDtypeStruct((N, D), f32),
            jax.ShapeDtypeStruct((N, D), f32),
            jax.ShapeDtypeStruct((N, D), f32),
            jax.ShapeDtypeStruct((N, D), f32),
            jax.ShapeDtypeStruct((2, G), jnp.int32),
        ),
        scratch_shapes=[pltpu.VMEM((8, G), jnp.int32)],
    )(h, u, gid2, Ws, bs)

    mesh = plsc.VectorSubcoreMesh(core_axis_name="c", subcore_axis_name="s")
    q_ = pl.kernel(
        _sc_gather_q_body,
        out_type=jax.ShapeDtypeStruct((E, D), f32),
        mesh=mesh,
        scratch_types=[
            pltpu.VMEM((_CB,), jnp.int32),
            pltpu.VMEM((_CB,), jnp.int32),
            pltpu.VMEM((_CB, D), f32),
            pltpu.VMEM((_CB, D), f32),
            pltpu.SemaphoreType.DMA,
        ],
    )(src, dst, p_, ah_)
    r_ = pl.kernel(
        _sc_gather_r_body,
        out_type=jax.ShapeDtypeStruct((E, D), f32),
        mesh=mesh,
        scratch_types=[
            pltpu.VMEM((_CB,), jnp.int32),
            pltpu.VMEM((_CB, D), f32),
            pltpu.SemaphoreType.DMA,
        ],
    )(src, eh_)

    nblk = E // _EB
    pre_, ac_ = pl.pallas_call(
        functools.partial(_edge_pass1_body, nblk=nblk, E=float(E)),
        grid=(nblk,),
        in_specs=[
            pl.BlockSpec((_EB, D), lambda i: (i, 0)),
            pl.BlockSpec((_EB, D), lambda i: (i, 0)),
            pl.BlockSpec((9, D, D), lambda i: (0, 0, 0)),
            pl.BlockSpec((9, D), lambda i: (0, 0)),
            pl.BlockSpec((3, D), lambda i: (0, 0)),
            pl.BlockSpec((3, D), lambda i: (0, 0)),
        ],
        out_specs=(
            pl.BlockSpec((_EB, D), lambda i: (i, 0)),
            pl.BlockSpec((2, D), lambda i: (0, 0)),
        ),
        out_shape=(
            jax.ShapeDtypeStruct((E, D), f32),
            jax.ShapeDtypeStruct((2, D), f32),
        ),
        scratch_shapes=[pltpu.VMEM((8, D), f32)],
    )(e, q_, Ws, bs, bn_gamma, bn_beta)

    stf = starts_
    enew_, v_, sg_, cnte_ = pl.pallas_call(
        functools.partial(_edge_pass2_body, nblk=nblk),
        grid=(nblk,),
        in_specs=[
            pl.BlockSpec((_EB, D), lambda i: (i, 0)),
            pl.BlockSpec((_EB, D), lambda i: (i, 0)),
            pl.BlockSpec((_EB, D), lambda i: (i, 0)),
            pl.BlockSpec((_EB, 1), lambda i: (i, 0)),
            pl.BlockSpec((2, D), lambda i: (0, 0)),
            pl.BlockSpec((2, G), lambda i: (0, 0)),
        ],
        out_specs=(
            pl.BlockSpec((_EB, D), lambda i: (i, 0)),
            pl.BlockSpec((_EB, 2 * D), lambda i: (i, 0)),
            pl.BlockSpec((G, D), lambda i: (0, 0)),
            pl.BlockSpec((G, D), lambda i: (0, 0)),
        ),
        out_shape=(
            jax.ShapeDtypeStruct((E, D), f32),
            jax.ShapeDtypeStruct((E, 2 * D), f32),
            jax.ShapeDtypeStruct((G, D), f32),
            jax.ShapeDtypeStruct((G, D), f32),
        ),
        scratch_shapes=[pltpu.VMEM((G, D), f32), pltpu.VMEM((G, D), f32)],
    )(pre_, e, r_, dst2, ac_, stf)

    nd_ = pl.kernel(
        _sc_scatter_body,
        out_type=jax.ShapeDtypeStruct((N, 2 * D), f32),
        mesh=mesh,
        scratch_types=[
            pltpu.VMEM((_CB,), jnp.int32),
            pltpu.VMEM((_CB, D), f32),
            pltpu.VMEM((104, D), f32),
            pltpu.VMEM_SHARED((N, D), f32),
            pltpu.SemaphoreType.DMA,
        ],
    )(v_, dst)

    hpre_, ac0_ = pl.pallas_call(
        functools.partial(_node_pass1_body, nblk=nnblk, N=float(N)),
        grid=(nnblk,),
        in_specs=[
            pl.BlockSpec((NB, 2 * D), lambda i: (i, 0)),
            pl.BlockSpec((NB, D), lambda i: (i, 0)),
            pl.BlockSpec((3, D), lambda i: (0, 0)),
            pl.BlockSpec((3, D), lambda i: (0, 0)),
        ],
        out_specs=(
            pl.BlockSpec((NB, D), lambda i: (i, 0)),
            pl.BlockSpec((2, D), lambda i: (0, 0)),
        ),
        out_shape=(
            jax.ShapeDtypeStruct((N, D), f32),
            jax.ShapeDtypeStruct((2, D), f32),
        ),
        scratch_shapes=[pltpu.VMEM((8, D), f32)],
    )(nd_, dfh_, bn_gamma, bn_beta)

    hn_, sh_, cntn_ = pl.pallas_call(
        functools.partial(_node_pass2_body, nblk=nnblk),
        grid=(nnblk,),
        in_specs=[
            pl.BlockSpec((NB, D), lambda i: (i, 0)),
            pl.BlockSpec((NB, D), lambda i: (i, 0)),
            pl.BlockSpec((NB, 1), lambda i: (i, 0)),
            pl.BlockSpec((2, D), lambda i: (0, 0)),
        ],
        out_specs=(
            pl.BlockSpec((NB, D), lambda i: (i, 0)),
            pl.BlockSpec((G, D), lambda i: (0, 0)),
            pl.BlockSpec((G, D), lambda i: (0, 0)),
        ),
        out_shape=(
            jax.ShapeDtypeStruct((N, D), f32),
            jax.ShapeDtypeStruct((G, D), f32),
            jax.ShapeDtypeStruct((G, D), f32),
        ),
        scratch_shapes=[pltpu.VMEM((G, D), f32), pltpu.VMEM((G, D), f32)],
    )(hpre_, h, gid2, ac0_)

    un_ = pl.pallas_call(
        functools.partial(_global_body, E=float(E), G=float(G)),
        out_shape=jax.ShapeDtypeStruct((G, D), f32),
    )(u, sg_, cnte_, sh_, cntn_, Ws, bs, bn_gamma, bn_beta)

    return (hn_, enew_, un_)


# speedup vs baseline: 3.8377x; 1.2521x over previous
import functools

import jax
import jax.numpy as jnp
from jax import lax
from jax.experimental import pallas as pl
from jax.experimental.pallas import tpu as pltpu
from jax.experimental.pallas import tpu_sc as plsc

_PREC = lax.Precision.HIGHEST

_D = 128
_G = 64
_EB = 1280
_CB = 128
_NW = 32
_NS = 16


def _node_proj_body(h_ref, u_ref, gid_ref, ws_ref, bs_ref,
                    p_ref, ah_ref, eh_ref, dfh_ref, starts_ref,
                    stacc_ref, *, nblk):
    i = pl.program_id(0)

    @pl.when(i == 0)
    def _():
        stacc_ref[...] = jnp.zeros_like(stacc_ref)

    h = h_ref[...]
    u = u_ref[...]
    ah = jnp.dot(h, ws_ref[0], precision=_PREC) + bs_ref[0]
    cu = jnp.dot(u, ws_ref[2], precision=_PREC) + bs_ref[2]
    dh = jnp.dot(h, ws_ref[3], precision=_PREC) + bs_ref[3]
    eh = jnp.dot(h, ws_ref[4], precision=_PREC) + bs_ref[4]
    fu = jnp.dot(u, ws_ref[5], precision=_PREC) + bs_ref[5]
    gid = gid_ref[...]
    giota = lax.broadcasted_iota(jnp.int32, (1, _G), 1)
    oh = (gid == giota).astype(jnp.float32)
    cu_atom = jnp.dot(oh, cu, precision=_PREC)
    fu_atom = jnp.dot(oh, fu, precision=_PREC)
    p_ref[...] = ah + cu_atom
    ah_ref[...] = ah
    eh_ref[...] = eh
    dfh_ref[...] = dh + fu_atom
    lt = (gid < giota).astype(jnp.int32)
    le = (gid <= giota).astype(jnp.int32)
    stacc_ref[0:1, :] += jnp.sum(lt, axis=0, keepdims=True)
    stacc_ref[1:2, :] += jnp.sum(le, axis=0, keepdims=True)

    @pl.when(i == nblk - 1)
    def _():
        starts_ref[...] = stacc_ref[0:2, :]


_CB2 = 80


def _sc_gather_q_body(src_hbm, dst_hbm, p_hbm, ah_hbm,
                      q_hbm,
                      src_all, dst_all, bufp0, bufp1, bufa0, bufa1,
                      semp0, semp1, sema0, sema1):
    E = src_hbm.shape[0]
    epw = E // _NW
    nch = epw // _CB2
    c = lax.axis_index("c")
    s = lax.axis_index("s")
    w = s * 2 + c
    base_w = w * epw
    bufp = (bufp0, bufp1)
    bufa = (bufa0, bufa1)
    semp = (semp0, semp1)
    sema = (sema0, sema1)

    pltpu.sync_copy(src_hbm.at[pl.ds(base_w, epw)], src_all)
    pltpu.sync_copy(dst_hbm.at[pl.ds(base_w, epw)], dst_all)

    def _issue(k, ci):
        isl = pl.ds(ci * _CB2, _CB2)
        pltpu.async_copy(p_hbm.at[src_all.at[isl]], bufp[k], semp[k])
        pltpu.async_copy(ah_hbm.at[dst_all.at[isl]], bufa[k], sema[k])

    def _wait(k, ci):
        isl = pl.ds(ci * _CB2, _CB2)
        pltpu.make_async_copy(p_hbm.at[src_all.at[isl]], bufp[k], semp[k]).wait()
        pltpu.make_async_copy(ah_hbm.at[dst_all.at[isl]], bufa[k], sema[k]).wait()

    _issue(0, 0)

    def trip(tt, carry):
        for pos in range(2):
            ci = 2 * tt + pos

            @pl.when(ci < nch)
            def _():
                @pl.when(ci + 1 < nch)
                def _():
                    _issue((pos + 1) % 2, ci + 1)

                _wait(pos, ci)

                def add_row(r, carry2):
                    for cc in range(_D // 16):
                        sl = pl.ds(cc * 16, 16)
                        bufp[pos][r, sl] = bufp[pos][r, sl] + bufa[pos][r, sl]
                    return carry2

                lax.fori_loop(0, _CB2, add_row, 0)
                pltpu.sync_copy(bufp[pos], q_hbm.at[pl.ds(base_w + ci * _CB2, _CB2)])

        return carry

    lax.fori_loop(0, (nch + 1) // 2, trip, 0)


def _sc_gather_r_body(src_hbm, eh_hbm, r_hbm,
                      src_all, bufe0, bufe1, seme0, seme1):
    E = src_hbm.shape[0]
    epw = E // _NW
    nch = epw // _CB2
    c = lax.axis_index("c")
    s = lax.axis_index("s")
    w = s * 2 + c
    base_w = w * epw
    bufe = (bufe0, bufe1)
    seme = (seme0, seme1)

    pltpu.sync_copy(src_hbm.at[pl.ds(base_w, epw)], src_all)

    def _issue(k, ci):
        isl = pl.ds(ci * _CB2, _CB2)
        pltpu.async_copy(eh_hbm.at[src_all.at[isl]], bufe[k], seme[k])

    def _wait(k, ci):
        isl = pl.ds(ci * _CB2, _CB2)
        pltpu.make_async_copy(eh_hbm.at[src_all.at[isl]], bufe[k], seme[k]).wait()

    _issue(0, 0)

    def trip(tt, carry):
        for pos in range(2):
            ci = 2 * tt + pos

            @pl.when(ci < nch)
            def _():
                @pl.when(ci + 1 < nch)
                def _():
                    _issue((pos + 1) % 2, ci + 1)

                _wait(pos, ci)
                pltpu.sync_copy(bufe[pos], r_hbm.at[pl.ds(base_w + ci * _CB2, _CB2)])

        return carry

    lax.fori_loop(0, (nch + 1) // 2, trip, 0)


def _edge_pass1_body(e_ref, q_ref, ws_ref, bs_ref, g_ref, b_ref,
                     pre_ref, ac_ref, acc_ref, *, nblk, E):
    i = pl.program_id(0)

    @pl.when(i == 0)
    def _():
        acc_ref[...] = jnp.zeros_like(acc_ref)

    pre = jnp.dot(e_ref[...], ws_ref[1], precision=_PREC) + bs_ref[1] + q_ref[...]
    pre_ref[...] = pre
    acc_ref[0:1, :] += jnp.sum(pre, axis=0, keepdims=True)
    acc_ref[1:2, :] += jnp.sum(pre * pre, axis=0, keepdims=True)

    @pl.when(i == nblk - 1)
    def _():
        m = acc_ref[0:1, :] / E
        v = acc_ref[1:2, :] / E - m * m
        a = g_ref[1:2, :] * lax.rsqrt(v + 1e-5)
        ac_ref[0:1, :] = a
        ac_ref[1:2, :] = b_ref[1:2, :] - m * a


def _edge_pass2_body(pre_ref, e_ref, r_ref, dst_ref, ac_ref, st_ref,
                     enew_ref, v_ref, sg_ref, cnte_ref,
                     sgacc_ref, cntacc_ref, *, nblk):
    i = pl.program_id(0)

    @pl.when(i == 0)
    def _():
        sgacc_ref[...] = jnp.zeros_like(sgacc_ref)
        cntacc_ref[...] = jnp.zeros_like(cntacc_ref)

    a = ac_ref[0:1, :]
    c = ac_ref[1:2, :]
    en = e_ref[...] + jnp.maximum(pre_ref[...] * a + c, 0.0)
    enew_ref[...] = en
    sig = 1.0 / (1.0 + jnp.exp(-en))
    v_ref[:, 0:_D] = sig * r_ref[...]
    v_ref[:, _D:2 * _D] = sig
    dstb = dst_ref[...]
    ge0 = (dstb >= st_ref[0:1, :]).astype(jnp.float32)
    ge1 = (dstb >= st_ref[1:2, :]).astype(jnp.float32)
    oh = ge0 - ge1
    dn = (((0,), (0,)), ((), ()))
    sgacc_ref[...] += lax.dot_general(oh, en, dn, precision=_PREC)
    cntacc_ref[...] += lax.dot_general(oh, jnp.ones_like(en), dn, precision=_PREC)

    @pl.when(i == nblk - 1)
    def _():
        sg_ref[...] = sgacc_ref[...]
        cnte_ref[...] = cntacc_ref[...]


def _sc_scatter_body(vv_hbm, dst_hbm, nd_hbm, dst_v, zbuf_idx, vbuf, vbufb,
                     zbuf, acc, semi0, semi1, semv0, semv1):
    semi = (semi0, semi1)
    semv = (semv0, semv1)
    E = dst_hbm.shape[0]
    N = nd_hbm.shape[0]
    nchunks = E // _CB
    ntiles = N // 8
    tiles_per_sub = ntiles // _NS
    rem = ntiles - tiles_per_sub * _NS
    zrows = zbuf.shape[0]
    nz = (tiles_per_sub * 8) // zrows
    c = lax.axis_index("c")
    s = lax.axis_index("s")
    t0 = s * tiles_per_sub + jnp.minimum(s, rem)
    extra = s < rem
    r0 = t0 * 8

    def zrow(r, carry):
        for cc in range(_D // 16):
            zbuf[r, pl.ds(cc * 16, 16)] = jnp.zeros((16,), jnp.float32)
        return carry

    lax.fori_loop(0, zrows, zrow, 0)

    def zcp(j, carry):
        pltpu.sync_copy(zbuf, acc.at[pl.ds(r0 + j * zrows, zrows)])
        return carry

    lax.fori_loop(0, nz, zcp, 0)

    @pl.when(extra)
    def _():
        pltpu.sync_copy(zbuf.at[pl.ds(0, 8)], acc.at[pl.ds(r0 + nz * zrows, 8)])

    plsc.subcore_barrier()

    eps = E // _NS
    nch = eps // _CB2
    base_s = s * eps
    dst_v2 = (dst_v, zbuf_idx)
    vbuf2 = (vbuf, vbufb)

    def _issue(k, ci):
        b = base_s + ci * _CB2
        pltpu.async_copy(dst_hbm.at[pl.ds(b, _CB2)], dst_v2[k], semi[k])
        pltpu.async_copy(vv_hbm.at[pl.ds(b, _CB2), pl.ds(c * _D, _D)],
                         vbuf2[k], semv[k])

    def _wait(k, ci):
        b = base_s + ci * _CB2
        pltpu.make_async_copy(dst_hbm.at[pl.ds(b, _CB2)], dst_v2[k], semi[k]).wait()
        pltpu.make_async_copy(vv_hbm.at[pl.ds(b, _CB2), pl.ds(c * _D, _D)],
                              vbuf2[k], semv[k]).wait()

    _issue(0, 0)

    def trip(tt, carry):
        for pos in range(2):
            ci = 2 * tt + pos

            @pl.when(ci < nch)
            def _():
                @pl.when(ci + 1 < nch)
                def _():
                    _issue((pos + 1) % 2, ci + 1)

                _wait(pos, ci)
                pltpu.sync_copy(vbuf2[pos], acc.at[dst_v2[pos]], add=True)

        return carry

    lax.fori_loop(0, (nch + 1) // 2, trip, 0)
    plsc.subcore_barrier()

    def dump(j, carry):
        rr = r0 + j * zrows
        pltpu.sync_copy(acc.at[pl.ds(rr, zrows)],
                        nd_hbm.at[pl.ds(rr, zrows), pl.ds(c * _D, _D)])
        return carry

    lax.fori_loop(0, nz, dump, 0)

    @pl.when(extra)
    def _():
        rr = r0 + nz * zrows
        pltpu.sync_copy(acc.at[pl.ds(rr, 8)],
                        nd_hbm.at[pl.ds(rr, 8), pl.ds(c * _D, _D)])


def _node_pass1_body(nd_ref, dfh_ref, g_ref, b_ref,
                     hpre_ref, ac_ref, acc_ref, *, nblk, N):
    i = pl.program_id(0)

    @pl.when(i == 0)
    def _():
        acc_ref[...] = jnp.zeros_like(acc_ref)

    num = nd_ref[:, 0:_D]
    den = nd_ref[:, _D:2 * _D]
    hpre = dfh_ref[...] + num / (den + 1e-6)
    hpre_ref[...] = hpre
    acc_ref[0:1, :] += jnp.sum(hpre, axis=0, keepdims=True)
    acc_ref[1:2, :] += jnp.sum(hpre * hpre, axis=0, keepdims=True)

    @pl.when(i == nblk - 1)
    def _():
        m = acc_ref[0:1, :] / N
        v = acc_ref[1:2, :] / N - m * m
        a = g_ref[0:1, :] * lax.rsqrt(v + 1e-5)
        ac_ref[0:1, :] = a
        ac_ref[1:2, :] = b_ref[0:1, :] - m * a


def _node_pass2_body(hpre_ref, h_ref, gid_ref, ac_ref,
                     hn_ref, sh_ref, cntn_ref, shacc_ref, cntacc_ref, *, nblk):
    i = pl.program_id(0)

    @pl.when(i == 0)
    def _():
        shacc_ref[...] = jnp.zeros_like(shacc_ref)
        cntacc_ref[...] = jnp.zeros_like(cntacc_ref)

    a = ac_ref[0:1, :]
    c = ac_ref[1:2, :]
    hn = h_ref[...] + jnp.maximum(hpre_ref[...] * a + c, 0.0)
    hn_ref[...] = hn
    gid = gid_ref[...]
    oh = (gid == lax.broadcasted_iota(jnp.int32, (1, _G), 1)).astype(jnp.float32)
    dn = (((0,), (0,)), ((), ()))
    shacc_ref[...] += lax.dot_general(oh, hn, dn, precision=_PREC)
    cntacc_ref[...] += lax.dot_general(oh, jnp.ones_like(hn), dn, precision=_PREC)

    @pl.when(i == nblk - 1)
    def _():
        sh_ref[...] = shacc_ref[...]
        cntn_ref[...] = cntacc_ref[...]


def _global_body(u_ref, sg_ref, cnte_ref, sh_ref, cntn_ref, ws_ref, bs_ref,
                 g_ref, b_ref, un_ref, *, E, G):
    u = u_ref[...]
    cntn = cntn_ref[...]
    mean_gh = (jnp.dot(sh_ref[...], ws_ref[6], precision=_PREC)
               + cntn * bs_ref[6]) / jnp.maximum(cntn, 1.0)
    mean_he = (jnp.dot(sg_ref[...], ws_ref[7], precision=_PREC)
               + cnte_ref[...] * bs_ref[7]) / E
    upre = (mean_gh + mean_he
            + jnp.dot(u, ws_ref[8], precision=_PREC) + bs_ref[8])
    m = jnp.sum(upre, axis=0, keepdims=True) / G
    v = jnp.sum(upre * upre, axis=0, keepdims=True) / G - m * m
    un_ref[...] = u + jnp.maximum(
        (upre - m) * lax.rsqrt(v + 1e-5) * g_ref[2:3, :] + b_ref[2:3, :], 0.0)


def kernel(h, e, u, Ws, bs, bn_gamma, bn_beta, edge_index, graph_ids):
    N, D = h.shape
    E = e.shape[0]
    G = u.shape[0]
    assert D == _D and G == _G and E % _EB == 0 and E % _CB == 0

    src = edge_index[0]
    dst = edge_index[1]
    gid2 = graph_ids.reshape(N, 1)
    dst2 = dst.reshape(E, 1)
    f32 = jnp.float32

    NB = 2000
    nnblk = N // NB
    p_, ah_, eh_, dfh_, starts_ = pl.pallas_call(
        functools.partial(_node_proj_body, nblk=nnblk),
        grid=(nnblk,),
        in_specs=[
            pl.BlockSpec((NB, D), lambda i: (i, 0)),
            pl.BlockSpec((G, D), lambda i: (0, 0)),
            pl.BlockSpec((NB, 1), lambda i: (i, 0)),
            pl.BlockSpec((9, D, D), lambda i: (0, 0, 0)),
            pl.BlockSpec((9, D), lambda i: (0, 0)),
        ],
        out_specs=(
            pl.BlockSpec((NB, D), lambda i: (i, 0)),
            pl.BlockSpec((NB, D), lambda i: (i, 0)),
            pl.BlockSpec((NB, D), lambda i: (i, 0)),
            pl.BlockSpec((NB, D), lambda i: (i, 0)),
            pl.BlockSpec((2, G), lambda i: (0, 0)),
        ),
        out_shape=(
            jax.ShapeDtypeStruct((N, D), f32),
            jax.ShapeDtypeStruct((N, D), f32),
            jax.ShapeDtypeStruct((N, D), f32),
            jax.ShapeDtypeStruct((N, D), f32),
            jax.ShapeDtypeStruct((2, G), jnp.int32),
        ),
        scratch_shapes=[pltpu.VMEM((8, G), jnp.int32)],
    )(h, u, gid2, Ws, bs)

    mesh = plsc.VectorSubcoreMesh(core_axis_name="c", subcore_axis_name="s")
    epw = E // _NW
    q_ = pl.kernel(
        _sc_gather_q_body,
        out_type=jax.ShapeDtypeStruct((E, D), f32),
        mesh=mesh,
        scratch_types=[
            pltpu.VMEM((epw,), jnp.int32),
            pltpu.VMEM((epw,), jnp.int32),
            pltpu.VMEM((_CB2, D), f32),
            pltpu.VMEM((_CB2, D), f32),
            pltpu.VMEM((_CB2, D), f32),
            pltpu.VMEM((_CB2, D), f32),
            pltpu.SemaphoreType.DMA,
            pltpu.SemaphoreType.DMA,
            pltpu.SemaphoreType.DMA,
            pltpu.SemaphoreType.DMA,
        ],
    )(src, dst, p_, ah_)
    r_ = pl.kernel(
        _sc_gather_r_body,
        out_type=jax.ShapeDtypeStruct((E, D), f32),
        mesh=mesh,
        scratch_types=[
            pltpu.VMEM((epw,), jnp.int32),
            pltpu.VMEM((_CB2, D), f32),
            pltpu.VMEM((_CB2, D), f32),
            pltpu.SemaphoreType.DMA,
            pltpu.SemaphoreType.DMA,
        ],
    )(src, eh_)

    nblk = E // _EB
    pre_, ac_ = pl.pallas_call(
        functools.partial(_edge_pass1_body, nblk=nblk, E=float(E)),
        grid=(nblk,),
        in_specs=[
            pl.BlockSpec((_EB, D), lambda i: (i, 0)),
            pl.BlockSpec((_EB, D), lambda i: (i, 0)),
            pl.BlockSpec((9, D, D), lambda i: (0, 0, 0)),
            pl.BlockSpec((9, D), lambda i: (0, 0)),
            pl.BlockSpec((3, D), lambda i: (0, 0)),
            pl.BlockSpec((3, D), lambda i: (0, 0)),
        ],
        out_specs=(
            pl.BlockSpec((_EB, D), lambda i: (i, 0)),
            pl.BlockSpec((2, D), lambda i: (0, 0)),
        ),
        out_shape=(
            jax.ShapeDtypeStruct((E, D), f32),
            jax.ShapeDtypeStruct((2, D), f32),
        ),
        scratch_shapes=[pltpu.VMEM((8, D), f32)],
    )(e, q_, Ws, bs, bn_gamma, bn_beta)

    stf = starts_
    enew_, v_, sg_, cnte_ = pl.pallas_call(
        functools.partial(_edge_pass2_body, nblk=nblk),
        grid=(nblk,),
        in_specs=[
            pl.BlockSpec((_EB, D), lambda i: (i, 0)),
            pl.BlockSpec((_EB, D), lambda i: (i, 0)),
            pl.BlockSpec((_EB, D), lambda i: (i, 0)),
            pl.BlockSpec((_EB, 1), lambda i: (i, 0)),
            pl.BlockSpec((2, D), lambda i: (0, 0)),
            pl.BlockSpec((2, G), lambda i: (0, 0)),
        ],
        out_specs=(
            pl.BlockSpec((_EB, D), lambda i: (i, 0)),
            pl.BlockSpec((_EB, 2 * D), lambda i: (i, 0)),
            pl.BlockSpec((G, D), lambda i: (0, 0)),
            pl.BlockSpec((G, D), lambda i: (0, 0)),
        ),
        out_shape=(
            jax.ShapeDtypeStruct((E, D), f32),
            jax.ShapeDtypeStruct((E, 2 * D), f32),
            jax.ShapeDtypeStruct((G, D), f32),
            jax.ShapeDtypeStruct((G, D), f32),
        ),
        scratch_shapes=[pltpu.VMEM((G, D), f32), pltpu.VMEM((G, D), f32)],
    )(pre_, e, r_, dst2, ac_, stf)

    nd_ = pl.kernel(
        _sc_scatter_body,
        out_type=jax.ShapeDtypeStruct((N, 2 * D), f32),
        mesh=mesh,
        scratch_types=[
            pltpu.VMEM((_CB2,), jnp.int32),
            pltpu.VMEM((_CB2,), jnp.int32),
            pltpu.VMEM((_CB2, D), f32),
            pltpu.VMEM((_CB2, D), f32),
            pltpu.VMEM((104, D), f32),
            pltpu.VMEM_SHARED((N, D), f32),
            pltpu.SemaphoreType.DMA,
            pltpu.SemaphoreType.DMA,
            pltpu.SemaphoreType.DMA,
            pltpu.SemaphoreType.DMA,
        ],
    )(v_, dst)

    hpre_, ac0_ = pl.pallas_call(
        functools.partial(_node_pass1_body, nblk=nnblk, N=float(N)),
        grid=(nnblk,),
        in_specs=[
            pl.BlockSpec((NB, 2 * D), lambda i: (i, 0)),
            pl.BlockSpec((NB, D), lambda i: (i, 0)),
            pl.BlockSpec((3, D), lambda i: (0, 0)),
            pl.BlockSpec((3, D), lambda i: (0, 0)),
        ],
        out_specs=(
            pl.BlockSpec((NB, D), lambda i: (i, 0)),
            pl.BlockSpec((2, D), lambda i: (0, 0)),
        ),
        out_shape=(
            jax.ShapeDtypeStruct((N, D), f32),
            jax.ShapeDtypeStruct((2, D), f32),
        ),
        scratch_shapes=[pltpu.VMEM((8, D), f32)],
    )(nd_, dfh_, bn_gamma, bn_beta)

    hn_, sh_, cntn_ = pl.pallas_call(
        functools.partial(_node_pass2_body, nblk=nnblk),
        grid=(nnblk,),
        in_specs=[
            pl.BlockSpec((NB, D), lambda i: (i, 0)),
            pl.BlockSpec((NB, D), lambda i: (i, 0)),
            pl.BlockSpec((NB, 1), lambda i: (i, 0)),
            pl.BlockSpec((2, D), lambda i: (0, 0)),
        ],
        out_specs=(
            pl.BlockSpec((NB, D), lambda i: (i, 0)),
            pl.BlockSpec((G, D), lambda i: (0, 0)),
            pl.BlockSpec((G, D), lambda i: (0, 0)),
        ),
        out_shape=(
            jax.ShapeDtypeStruct((N, D), f32),
            jax.ShapeDtypeStruct((G, D), f32),
            jax.ShapeDtypeStruct((G, D), f32),
        ),
        scratch_shapes=[pltpu.VMEM((G, D), f32), pltpu.VMEM((G, D), f32)],
    )(hpre_, h, gid2, ac0_)

    un_ = pl.pallas_call(
        functools.partial(_global_body, E=float(E), G=float(G)),
        out_shape=jax.ShapeDtypeStruct((G, D), f32),
    )(u, sg_, cnte_, sh_, cntn_, Ws, bs, bn_gamma, bn_beta)

    return (hn_, enew_, un_)
